# baseline (speedup 1.0000x reference)
import functools

import jax
import jax.numpy as jnp
from jax import lax
from jax.experimental import pallas as pl
from jax.experimental.pallas import tpu as pltpu
from jax.experimental.pallas import tpu_sc as plsc

_IMG_SIZE = 416.0
_NCLS = 80
_EPS = 1e-7
_B = 32
_NB = 20
_GRIDS = (13, 26, 52)
_ANCHORS = [[[116.0, 90.0], [156.0, 198.0], [373.0, 326.0]],
            [[30.0, 61.0], [62.0, 45.0], [59.0, 119.0]],
            [[10.0, 13.0], [16.0, 30.0], [33.0, 23.0]]]
_AW = [[a[0] / (_IMG_SIZE / g) for a in _ANCHORS[s]] for s, g in enumerate(_GRIDS)]
_AH = [[a[1] / (_IMG_SIZE / g) for a in _ANCHORS[s]] for s, g in enumerate(_GRIDS)]
_NCELLS = tuple(_B * 3 * g * g for g in _GRIDS)
_NSTEP = 39


def _best_anchor(wg, hg, s):
    iou = []
    for a in range(3):
        inter = jnp.minimum(wg, _AW[s][a]) * jnp.minimum(hg, _AH[s][a])
        union = _AW[s][a] * _AH[s][a] + wg + hg - inter
        iou.append(jnp.where(union > 0, inter / union, 0.0))
    best = jnp.where(iou[1] > iou[0], jnp.full(wg.shape, 1, jnp.int32),
                     jnp.full(wg.shape, 0, jnp.int32))
    best = jnp.where(iou[2] > jnp.maximum(iou[0], iou[1]),
                     jnp.full(wg.shape, 2, jnp.int32), best)
    return best


def _dense_asm_body(boxes_ref, labels_ref, rows_ref, meta_ref,
                    p1ref, p2ref, p3ref,
                    o_total, o_coord, o_obj, o_noobj, o_class, acc_ref):
    i = pl.program_id(0)
    for s, pref in enumerate((p1ref, p2ref, p3ref)):
        p = pref[:, :, 4:5]
        pc = jnp.clip(p, _EPS, 1.0 - _EPS)
        bsum = jnp.sum(-jnp.log(1.0 - pc))
        prev = jnp.where(i == 0, 0.0, acc_ref[s])
        acc_ref[s] = prev + bsum

    @pl.when(i == _NSTEP - 1)
    def _():
        dense = (acc_ref[0], acc_ref[1], acc_ref[2])
        _asm_core(boxes_ref, labels_ref, rows_ref, meta_ref, dense,
                  o_total, o_coord, o_obj, o_noobj, o_class)


def _dense_asm(boxes, labels, rows, meta, p1t, p2t, p3t):
    sd = jax.ShapeDtypeStruct((1, 1), jnp.float32)
    small = [pl.BlockSpec((_B, _NB, 4), lambda i: (0, 0, 0)),
             pl.BlockSpec((_B, _NB), lambda i: (0, 0)),
             pl.BlockSpec((3, _B, 32, 85), lambda i: (0, 0, 0, 0)),
             pl.BlockSpec((3, _B, 4, 32), lambda i: (0, 0, 0, 0))]
    return pl.pallas_call(
        _dense_asm_body,
        grid=(_NSTEP,),
        in_specs=small +
                 [pl.BlockSpec((3 * g * g // _NSTEP, _B, 85),
                               lambda i: (i, 0, 0))
                  for g in _GRIDS],
        out_specs=[pl.BlockSpec((1, 1), lambda i: (0, 0))] * 5,
        out_shape=(sd, sd, sd, sd, sd),
        scratch_shapes=[pltpu.SMEM((4,), jnp.float32)],
        compiler_params=pltpu.CompilerParams(
            dimension_semantics=("arbitrary",)),
    )(boxes, labels, rows, meta, p1t, p2t, p3t)


def _sc_gather(boxes_t, p1t, p2t, p3t):
    mesh = plsc.VectorSubcoreMesh(core_axis_name="c", subcore_axis_name="s")

    @functools.partial(
        pl.kernel,
        mesh=mesh,
        out_type=(jax.ShapeDtypeStruct((3, _B, 32, 85), jnp.float32),
                  jax.ShapeDtypeStruct((3, _B, 4, 32), jnp.int32)),
        scratch_types=[pltpu.VMEM((4, 32), jnp.float32),
                       pltpu.VMEM((4, 32), jnp.int32),
                       pltpu.VMEM((3, 32, 85), jnp.float32),
                       pltpu.SemaphoreType.DMA],
        compiler_params=pltpu.CompilerParams(use_tc_tiling_on_sc=True),
    )
    def body(boxes_hbm, p1, p2, p3, rows_out, meta_out, bx_v, idx_v, rows_v,
             sem):
        b = lax.axis_index("s") * 2 + lax.axis_index("c")
        pltpu.sync_copy(boxes_hbm.at[b], bx_v)
        tabs = (p1, p2, p3)
        copies = []
        for s in range(3):
            g = _GRIDS[s]
            gf = jnp.float32(g)
            for k in range(2):
                xs = bx_v[0, pl.ds(k * 16, 16)]
                ys = bx_v[1, pl.ds(k * 16, 16)]
                ws = bx_v[2, pl.ds(k * 16, 16)]
                hs = bx_v[3, pl.ds(k * 16, 16)]
                fx = xs * gf
                fy = ys * gf
                gx = fx.astype(jnp.int32)
                gy = fy.astype(jnp.int32)
                gxc = jnp.minimum(gx, g - 1)
                gyc = jnp.minimum(gy, g - 1)
                best = _best_anchor(ws * gf, hs * gf, s)
                cell = (best * g + gyc) * g + gxc
                idx_v[0, pl.ds(k * 16, 16)] = cell
                idx_v[1, pl.ds(k * 16, 16)] = best
                idx_v[2, pl.ds(k * 16, 16)] = gyc
                idx_v[3, pl.ds(k * 16, 16)] = gxc
                for j in range(16 if k == 0 else _NB - 16):
                    copies.append(pltpu.async_copy(
                        tabs[s].at[cell[j], b],
                        rows_v.at[s, k * 16 + j], sem))
            pltpu.sync_copy(idx_v, meta_out.at[s, b])
        for cp in copies:
            cp.wait()
        for s in range(3):
            pltpu.sync_copy(rows_v.at[s], rows_out.at[s, b])

    return body(boxes_t, p1t, p2t, p3t)


def _asm_core(boxes_ref, labels_ref, rows_ref, meta_ref, dense,
              o_total, o_coord, o_obj, o_noobj, o_class):
    coord_loss = jnp.float32(0.0)
    obj_loss = jnp.float32(0.0)
    noobj_loss = jnp.float32(0.0)
    class_loss = jnp.float32(0.0)
    labels = labels_ref[...]
    for s in range(3):
        g = _GRIDS[s]
        gf = jnp.float32(g)
        x = boxes_ref[:, :, 0]
        y = boxes_ref[:, :, 1]
        w = boxes_ref[:, :, 2]
        h = boxes_ref[:, :, 3]
        fx = x * gf
        fy = y * gf
        gx = fx.astype(jnp.int32)
        gy = fy.astype(jnp.int32)
        valid = (gx < g) & (gy < g)
        tx = fx - gx.astype(jnp.float32)
        ty = fy - gy.astype(jnp.float32)
        wg = w * gf
        hg = h * gf
        best = _best_anchor(wg, hg, s)
        awb = jnp.where(best == 1, _AW[s][1], _AW[s][0])
        awb = jnp.where(best == 2, _AW[s][2], awb)
        ahb = jnp.where(best == 1, _AH[s][1], _AH[s][0])
        ahb = jnp.where(best == 2, _AH[s][2], ahb)
        tw = wg / awb
        th = hg / ahb
        key = meta_ref[s, :, 0, :_NB]
        eq = key[:, :, None] == key[:, None, :]
        ii = lax.broadcasted_iota(jnp.int32, (_B, _NB, _NB), 1)
        jj = lax.broadcasted_iota(jnp.int32, (_B, _NB, _NB), 2)
        conflict = jnp.any(eq & (jj > ii) & valid[:, None, :], axis=-1)
        winner = valid & ~conflict
        wm = winner.astype(jnp.float32)
        n_obj = jnp.sum(wm)
        rows = rows_ref[s][:, :_NB, :]
        px = rows[:, :, 0]
        py = rows[:, :, 1]
        pw = rows[:, :, 2]
        ph = rows[:, :, 3]
        pobj = rows[:, :, 4]
        pcls = rows[:, :, 5:]
        n_div = jnp.maximum(n_obj * 2.0, 1.0)
        mse_xy = jnp.sum(wm * ((px - tx) ** 2 + (py - ty) ** 2)) / n_div
        mse_wh = jnp.sum(wm * ((jnp.sqrt(pw) - jnp.sqrt(tw)) ** 2
                               + (jnp.sqrt(ph) - jnp.sqrt(th)) ** 2)) / n_div
        has_obj = (n_obj > 0).astype(jnp.float32)
        coord_loss = coord_loss + has_obj * (mse_xy + mse_wh)
        pobj_c = jnp.clip(pobj, _EPS, 1.0 - _EPS)
        obj_loss = obj_loss + jnp.sum(wm * (-jnp.log(pobj_c))) / jnp.maximum(n_obj, 1.0)
        corr = jnp.sum(wm * (-jnp.log(1.0 - pobj_c)))
        n_noobj = _NCELLS[s] - n_obj
        noobj_loss = noobj_loss + (dense[s] - corr) / jnp.maximum(n_noobj, 1.0)
        cidx = lax.broadcasted_iota(jnp.int32, (_B, _NB, _NCLS), 2)
        onehot = (cidx == labels[:, :, None]).astype(jnp.float32)
        pc = jnp.clip(pcls, _EPS, 1.0 - _EPS)
        bce = -(onehot * jnp.log(pc) + (1.0 - onehot) * jnp.log(1.0 - pc))
        class_loss = class_loss + has_obj * (
            jnp.sum(wm[:, :, None] * bce) / jnp.maximum(n_obj * _NCLS, 1.0))
    total = (5.0 * coord_loss + obj_loss + 0.5 * noobj_loss + class_loss) / _B
    o_total[...] = jnp.full((1, 1), total, jnp.float32)
    o_coord[...] = jnp.full((1, 1), coord_loss / _B, jnp.float32)
    o_obj[...] = jnp.full((1, 1), obj_loss / _B, jnp.float32)
    o_noobj[...] = jnp.full((1, 1), noobj_loss / _B, jnp.float32)
    o_class[...] = jnp.full((1, 1), class_loss / _B, jnp.float32)


def kernel(pred_s1, pred_s2, pred_s3, boxes, labels):
    p1t, p2t, p3t = (p.transpose(1, 2, 3, 0, 4).reshape(-1, _B, 85)
                     for p in (pred_s1, pred_s2, pred_s3))
    boxes_t = jnp.pad(boxes, ((0, 0), (0, 32 - _NB), (0, 0)),
                      mode="edge").transpose(0, 2, 1)
    rows, meta = _sc_gather(boxes_t, p1t, p2t, p3t)
    t, c, o, n, cl = _dense_asm(boxes, labels.astype(jnp.int32), rows, meta,
                                p1t, p2t, p3t)
    return (t.reshape(()), c.reshape(()), o.reshape(()),
            n.reshape(()), cl.reshape(()))


# speedup vs baseline: 1.0256x; 1.0256x over previous
import functools

import jax
import jax.numpy as jnp
from jax import lax
from jax.experimental import pallas as pl
from jax.experimental.pallas import tpu as pltpu
from jax.experimental.pallas import tpu_sc as plsc

_IMG_SIZE = 416.0
_NCLS = 80
_EPS = 1e-7
_B = 32
_NB = 20
_GRIDS = (13, 26, 52)
_ANCHORS = [[[116.0, 90.0], [156.0, 198.0], [373.0, 326.0]],
            [[30.0, 61.0], [62.0, 45.0], [59.0, 119.0]],
            [[10.0, 13.0], [16.0, 30.0], [33.0, 23.0]]]
_AW = [[a[0] / (_IMG_SIZE / g) for a in _ANCHORS[s]] for s, g in enumerate(_GRIDS)]
_AH = [[a[1] / (_IMG_SIZE / g) for a in _ANCHORS[s]] for s, g in enumerate(_GRIDS)]
_NCELLS = tuple(_B * 3 * g * g for g in _GRIDS)
_NSTEP = 39


def _best_anchor(wg, hg, s):
    iou = []
    for a in range(3):
        inter = jnp.minimum(wg, _AW[s][a]) * jnp.minimum(hg, _AH[s][a])
        union = _AW[s][a] * _AH[s][a] + wg + hg - inter
        iou.append(jnp.where(union > 0, inter / union, 0.0))
    best = jnp.where(iou[1] > iou[0], jnp.full(wg.shape, 1, jnp.int32),
                     jnp.full(wg.shape, 0, jnp.int32))
    best = jnp.where(iou[2] > jnp.maximum(iou[0], iou[1]),
                     jnp.full(wg.shape, 2, jnp.int32), best)
    return best


def _dense_body(p1ref, p2ref, p3ref, o1ref, o2ref, o3ref):
    for pref, oref in ((p1ref, o1ref), (p2ref, o2ref), (p3ref, o3ref)):
        p = pref[:, :, 4:5]
        pc = jnp.clip(p, _EPS, 1.0 - _EPS)
        oref[...] = jnp.full((1, 1, 1), jnp.sum(-jnp.log(1.0 - pc)),
                             jnp.float32)


def _dense_sum(p1t, p2t, p3t):
    sd = jax.ShapeDtypeStruct((_NSTEP, 1, 1), jnp.float32)
    return pl.pallas_call(
        _dense_body,
        grid=(_NSTEP,),
        in_specs=[pl.BlockSpec((3 * g * g // _NSTEP, _B, 85),
                               lambda i: (i, 0, 0))
                  for g in _GRIDS],
        out_specs=[pl.BlockSpec((1, 1, 1), lambda i: (i, 0, 0))] * 3,
        out_shape=(sd, sd, sd),
        compiler_params=pltpu.CompilerParams(
            dimension_semantics=("parallel",)),
    )(p1t, p2t, p3t)


def _assembly(parts1, parts2, parts3, boxes, labels, rows, meta):
    sd = jax.ShapeDtypeStruct((1, 1), jnp.float32)

    def body(p1, p2, p3, boxes_ref, labels_ref, rows_ref, meta_ref, *outs):
        dense = (jnp.sum(p1[...]), jnp.sum(p2[...]), jnp.sum(p3[...]))
        _asm_core(boxes_ref, labels_ref, rows_ref, meta_ref, dense, *outs)

    return pl.pallas_call(
        body,
        out_shape=(sd, sd, sd, sd, sd),
    )(parts1, parts2, parts3, boxes, labels, rows, meta)


def _sc_gather(boxes_t, p1t, p2t, p3t):
    mesh = plsc.VectorSubcoreMesh(core_axis_name="c", subcore_axis_name="s")

    @functools.partial(
        pl.kernel,
        mesh=mesh,
        out_type=(jax.ShapeDtypeStruct((3, _B, 32, 85), jnp.float32),
                  jax.ShapeDtypeStruct((3, _B, 4, 32), jnp.int32)),
        scratch_types=[pltpu.VMEM((4, 32), jnp.float32),
                       pltpu.VMEM((4, 32), jnp.int32),
                       pltpu.VMEM((3, 32, 85), jnp.float32),
                       pltpu.SemaphoreType.DMA],
        compiler_params=pltpu.CompilerParams(use_tc_tiling_on_sc=True),
    )
    def body(boxes_hbm, p1, p2, p3, rows_out, meta_out, bx_v, idx_v, rows_v,
             sem):
        b = lax.axis_index("s") * 2 + lax.axis_index("c")
        pltpu.sync_copy(boxes_hbm.at[b], bx_v)
        tabs = (p1, p2, p3)
        copies = []
        for s in range(3):
            g = _GRIDS[s]
            gf = jnp.float32(g)
            for k in range(2):
                xs = bx_v[0, pl.ds(k * 16, 16)]
                ys = bx_v[1, pl.ds(k * 16, 16)]
                ws = bx_v[2, pl.ds(k * 16, 16)]
                hs = bx_v[3, pl.ds(k * 16, 16)]
                fx = xs * gf
                fy = ys * gf
                gx = fx.astype(jnp.int32)
                gy = fy.astype(jnp.int32)
                gxc = jnp.minimum(gx, g - 1)
                gyc = jnp.minimum(gy, g - 1)
                best = _best_anchor(ws * gf, hs * gf, s)
                cell = (best * g + gyc) * g + gxc
                idx_v[0, pl.ds(k * 16, 16)] = cell
                idx_v[1, pl.ds(k * 16, 16)] = best
                idx_v[2, pl.ds(k * 16, 16)] = gyc
                idx_v[3, pl.ds(k * 16, 16)] = gxc
                for j in range(16 if k == 0 else _NB - 16):
                    copies.append(pltpu.async_copy(
                        tabs[s].at[cell[j], b],
                        rows_v.at[s, k * 16 + j], sem))
            pltpu.sync_copy(idx_v, meta_out.at[s, b])
        for cp in copies:
            cp.wait()
        for s in range(3):
            pltpu.sync_copy(rows_v.at[s], rows_out.at[s, b])

    return body(boxes_t, p1t, p2t, p3t)


def _asm_core(boxes_ref, labels_ref, rows_ref, meta_ref, dense,
              o_total, o_coord, o_obj, o_noobj, o_class):
    coord_loss = jnp.float32(0.0)
    obj_loss = jnp.float32(0.0)
    noobj_loss = jnp.float32(0.0)
    class_loss = jnp.float32(0.0)
    labels = labels_ref[...]
    for s in range(3):
        g = _GRIDS[s]
        gf = jnp.float32(g)
        x = boxes_ref[:, :, 0]
        y = boxes_ref[:, :, 1]
        w = boxes_ref[:, :, 2]
        h = boxes_ref[:, :, 3]
        fx = x * gf
        fy = y * gf
        gx = fx.astype(jnp.int32)
        gy = fy.astype(jnp.int32)
        valid = (gx < g) & (gy < g)
        tx = fx - gx.astype(jnp.float32)
        ty = fy - gy.astype(jnp.float32)
        wg = w * gf
        hg = h * gf
        best = _best_anchor(wg, hg, s)
        awb = jnp.where(best == 1, _AW[s][1], _AW[s][0])
        awb = jnp.where(best == 2, _AW[s][2], awb)
        ahb = jnp.where(best == 1, _AH[s][1], _AH[s][0])
        ahb = jnp.where(best == 2, _AH[s][2], ahb)
        tw = wg / awb
        th = hg / ahb
        key = meta_ref[s, :, 0, :_NB]
        eq = key[:, :, None] == key[:, None, :]
        ii = lax.broadcasted_iota(jnp.int32, (_B, _NB, _NB), 1)
        jj = lax.broadcasted_iota(jnp.int32, (_B, _NB, _NB), 2)
        conflict = jnp.any(eq & (jj > ii) & valid[:, None, :], axis=-1)
        winner = valid & ~conflict
        wm = winner.astype(jnp.float32)
        n_obj = jnp.sum(wm)
        rows = rows_ref[s][:, :_NB, :]
        px = rows[:, :, 0]
        py = rows[:, :, 1]
        pw = rows[:, :, 2]
        ph = rows[:, :, 3]
        pobj = rows[:, :, 4]
        pcls = rows[:, :, 5:]
        n_div = jnp.maximum(n_obj * 2.0, 1.0)
        mse_xy = jnp.sum(wm * ((px - tx) ** 2 + (py - ty) ** 2)) / n_div
        mse_wh = jnp.sum(wm * ((jnp.sqrt(pw) - jnp.sqrt(tw)) ** 2
                               + (jnp.sqrt(ph) - jnp.sqrt(th)) ** 2)) / n_div
        has_obj = (n_obj > 0).astype(jnp.float32)
        coord_loss = coord_loss + has_obj * (mse_xy + mse_wh)
        pobj_c = jnp.clip(pobj, _EPS, 1.0 - _EPS)
        obj_loss = obj_loss + jnp.sum(wm * (-jnp.log(pobj_c))) / jnp.maximum(n_obj, 1.0)
        corr = jnp.sum(wm * (-jnp.log(1.0 - pobj_c)))
        n_noobj = _NCELLS[s] - n_obj
        noobj_loss = noobj_loss + (dense[s] - corr) / jnp.maximum(n_noobj, 1.0)
        cidx = lax.broadcasted_iota(jnp.int32, (_B, _NB, _NCLS), 2)
        onehot = (cidx == labels[:, :, None]).astype(jnp.float32)
        pc = jnp.clip(pcls, _EPS, 1.0 - _EPS)
        bce = -(onehot * jnp.log(pc) + (1.0 - onehot) * jnp.log(1.0 - pc))
        class_loss = class_loss + has_obj * (
            jnp.sum(wm[:, :, None] * bce) / jnp.maximum(n_obj * _NCLS, 1.0))
    total = (5.0 * coord_loss + obj_loss + 0.5 * noobj_loss + class_loss) / _B
    o_total[...] = jnp.full((1, 1), total, jnp.float32)
    o_coord[...] = jnp.full((1, 1), coord_loss / _B, jnp.float32)
    o_obj[...] = jnp.full((1, 1), obj_loss / _B, jnp.float32)
    o_noobj[...] = jnp.full((1, 1), noobj_loss / _B, jnp.float32)
    o_class[...] = jnp.full((1, 1), class_loss / _B, jnp.float32)


def kernel(pred_s1, pred_s2, pred_s3, boxes, labels):
    p1t, p2t, p3t = (p.transpose(1, 2, 3, 0, 4).reshape(-1, _B, 85)
                     for p in (pred_s1, pred_s2, pred_s3))
    boxes_t = jnp.pad(boxes, ((0, 0), (0, 32 - _NB), (0, 0)),
                      mode="edge").transpose(0, 2, 1)
    rows, meta = _sc_gather(boxes_t, p1t, p2t, p3t)
    parts1, parts2, parts3 = _dense_sum(p1t, p2t, p3t)
    t, c, o, n, cl = _assembly(parts1, parts2, parts3, boxes,
                               labels.astype(jnp.int32), rows, meta)
    return (t.reshape(()), c.reshape(()), o.reshape(()),
            n.reshape(()), cl.reshape(()))


# speedup vs baseline: 1.1988x; 1.1689x over previous
import functools

import jax
import jax.numpy as jnp
from jax import lax
from jax.experimental import pallas as pl
from jax.experimental.pallas import tpu as pltpu
from jax.experimental.pallas import tpu_sc as plsc

_IMG_SIZE = 416.0
_NCLS = 80
_EPS = 1e-7
_B = 32
_NB = 20
_GRIDS = (13, 26, 52)
_ANCHORS = [[[116.0, 90.0], [156.0, 198.0], [373.0, 326.0]],
            [[30.0, 61.0], [62.0, 45.0], [59.0, 119.0]],
            [[10.0, 13.0], [16.0, 30.0], [33.0, 23.0]]]
_AW = [[a[0] / (_IMG_SIZE / g) for a in _ANCHORS[s]] for s, g in enumerate(_GRIDS)]
_AH = [[a[1] / (_IMG_SIZE / g) for a in _ANCHORS[s]] for s, g in enumerate(_GRIDS)]
_NCELLS = tuple(_B * 3 * g * g for g in _GRIDS)
_NSTEP = 13


def _best_anchor(wg, hg, s):
    iou = []
    for a in range(3):
        inter = jnp.minimum(wg, _AW[s][a]) * jnp.minimum(hg, _AH[s][a])
        union = _AW[s][a] * _AH[s][a] + wg + hg - inter
        iou.append(jnp.where(union > 0, inter / union, 0.0))
    best = jnp.where(iou[1] > iou[0], jnp.full(wg.shape, 1, jnp.int32),
                     jnp.full(wg.shape, 0, jnp.int32))
    best = jnp.where(iou[2] > jnp.maximum(iou[0], iou[1]),
                     jnp.full(wg.shape, 2, jnp.int32), best)
    return best


def _dense_body(p1ref, p2ref, p3ref, o1ref, o2ref, o3ref):
    for pref, oref in ((p1ref, o1ref), (p2ref, o2ref), (p3ref, o3ref)):
        p = pref[:, :, 4:5]
        pc = jnp.clip(p, _EPS, 1.0 - _EPS)
        oref[...] = jnp.full((1, 1, 1), jnp.sum(-jnp.log(1.0 - pc)),
                             jnp.float32)


def _dense_sum(p1t, p2t, p3t):
    sd = jax.ShapeDtypeStruct((_NSTEP, 1, 1), jnp.float32)
    return pl.pallas_call(
        _dense_body,
        grid=(_NSTEP,),
        in_specs=[pl.BlockSpec((3 * g * g // _NSTEP, _B, 85),
                               lambda i: (i, 0, 0))
                  for g in _GRIDS],
        out_specs=[pl.BlockSpec((1, 1, 1), lambda i: (i, 0, 0))] * 3,
        out_shape=(sd, sd, sd),
        compiler_params=pltpu.CompilerParams(
            dimension_semantics=("parallel",)),
    )(p1t, p2t, p3t)


def _assembly(parts1, parts2, parts3, boxes, labels, rows, meta):
    sd = jax.ShapeDtypeStruct((1, 1), jnp.float32)

    def body(p1, p2, p3, boxes_ref, labels_ref, rows_ref, meta_ref, *outs):
        dense = (jnp.sum(p1[...]), jnp.sum(p2[...]), jnp.sum(p3[...]))
        _asm_core(boxes_ref, labels_ref, rows_ref, meta_ref, dense, *outs)

    return pl.pallas_call(
        body,
        out_shape=(sd, sd, sd, sd, sd),
    )(parts1, parts2, parts3, boxes, labels, rows, meta)


def _sc_gather(boxes_t, p1t, p2t, p3t):
    mesh = plsc.VectorSubcoreMesh(core_axis_name="c", subcore_axis_name="s")

    @functools.partial(
        pl.kernel,
        mesh=mesh,
        out_type=(jax.ShapeDtypeStruct((3, _B, 32, 85), jnp.float32),
                  jax.ShapeDtypeStruct((3, _B, 4, 32), jnp.int32)),
        scratch_types=[pltpu.VMEM((4, 32), jnp.float32),
                       pltpu.VMEM((4, 32), jnp.int32),
                       pltpu.VMEM((3, 32, 85), jnp.float32),
                       pltpu.SemaphoreType.DMA],
        compiler_params=pltpu.CompilerParams(use_tc_tiling_on_sc=True),
    )
    def body(boxes_hbm, p1, p2, p3, rows_out, meta_out, bx_v, idx_v, rows_v,
             sem):
        b = lax.axis_index("s") * 2 + lax.axis_index("c")
        pltpu.sync_copy(boxes_hbm.at[b], bx_v)
        tabs = (p1, p2, p3)
        copies = []
        for s in range(3):
            g = _GRIDS[s]
            gf = jnp.float32(g)
            for k in range(2):
                xs = bx_v[0, pl.ds(k * 16, 16)]
                ys = bx_v[1, pl.ds(k * 16, 16)]
                ws = bx_v[2, pl.ds(k * 16, 16)]
                hs = bx_v[3, pl.ds(k * 16, 16)]
                fx = xs * gf
                fy = ys * gf
                gx = fx.astype(jnp.int32)
                gy = fy.astype(jnp.int32)
                gxc = jnp.minimum(gx, g - 1)
                gyc = jnp.minimum(gy, g - 1)
                best = _best_anchor(ws * gf, hs * gf, s)
                cell = (best * g + gyc) * g + gxc
                idx_v[0, pl.ds(k * 16, 16)] = cell
                idx_v[1, pl.ds(k * 16, 16)] = best
                idx_v[2, pl.ds(k * 16, 16)] = gyc
                idx_v[3, pl.ds(k * 16, 16)] = gxc
                for j in range(16 if k == 0 else _NB - 16):
                    copies.append(pltpu.async_copy(
                        tabs[s].at[cell[j], b],
                        rows_v.at[s, k * 16 + j], sem))
            pltpu.sync_copy(idx_v, meta_out.at[s, b])
        for cp in copies:
            cp.wait()
        for s in range(3):
            pltpu.sync_copy(rows_v.at[s], rows_out.at[s, b])

    return body(boxes_t, p1t, p2t, p3t)


def _asm_core(boxes_ref, labels_ref, rows_ref, meta_ref, dense,
              o_total, o_coord, o_obj, o_noobj, o_class):
    coord_loss = jnp.float32(0.0)
    obj_loss = jnp.float32(0.0)
    noobj_loss = jnp.float32(0.0)
    class_loss = jnp.float32(0.0)
    labels = labels_ref[...]
    for s in range(3):
        g = _GRIDS[s]
        gf = jnp.float32(g)
        x = boxes_ref[:, :, 0]
        y = boxes_ref[:, :, 1]
        w = boxes_ref[:, :, 2]
        h = boxes_ref[:, :, 3]
        fx = x * gf
        fy = y * gf
        gx = fx.astype(jnp.int32)
        gy = fy.astype(jnp.int32)
        valid = (gx < g) & (gy < g)
        tx = fx - gx.astype(jnp.float32)
        ty = fy - gy.astype(jnp.float32)
        wg = w * gf
        hg = h * gf
        best = _best_anchor(wg, hg, s)
        awb = jnp.where(best == 1, _AW[s][1], _AW[s][0])
        awb = jnp.where(best == 2, _AW[s][2], awb)
        ahb = jnp.where(best == 1, _AH[s][1], _AH[s][0])
        ahb = jnp.where(best == 2, _AH[s][2], ahb)
        tw = wg / awb
        th = hg / ahb
        key = meta_ref[s, :, 0, :_NB]
        eq = key[:, :, None] == key[:, None, :]
        ii = lax.broadcasted_iota(jnp.int32, (_B, _NB, _NB), 1)
        jj = lax.broadcasted_iota(jnp.int32, (_B, _NB, _NB), 2)
        conflict = jnp.any(eq & (jj > ii) & valid[:, None, :], axis=-1)
        winner = valid & ~conflict
        wm = winner.astype(jnp.float32)
        n_obj = jnp.sum(wm)
        rows = rows_ref[s][:, :_NB, :]
        px = rows[:, :, 0]
        py = rows[:, :, 1]
        pw = rows[:, :, 2]
        ph = rows[:, :, 3]
        pobj = rows[:, :, 4]
        pcls = rows[:, :, 5:]
        n_div = jnp.maximum(n_obj * 2.0, 1.0)
        mse_xy = jnp.sum(wm * ((px - tx) ** 2 + (py - ty) ** 2)) / n_div
        mse_wh = jnp.sum(wm * ((jnp.sqrt(pw) - jnp.sqrt(tw)) ** 2
                               + (jnp.sqrt(ph) - jnp.sqrt(th)) ** 2)) / n_div
        has_obj = (n_obj > 0).astype(jnp.float32)
        coord_loss = coord_loss + has_obj * (mse_xy + mse_wh)
        pobj_c = jnp.clip(pobj, _EPS, 1.0 - _EPS)
        obj_loss = obj_loss + jnp.sum(wm * (-jnp.log(pobj_c))) / jnp.maximum(n_obj, 1.0)
        corr = jnp.sum(wm * (-jnp.log(1.0 - pobj_c)))
        n_noobj = _NCELLS[s] - n_obj
        noobj_loss = noobj_loss + (dense[s] - corr) / jnp.maximum(n_noobj, 1.0)
        cidx = lax.broadcasted_iota(jnp.int32, (_B, _NB, _NCLS), 2)
        onehot = (cidx == labels[:, :, None]).astype(jnp.float32)
        pc = jnp.clip(pcls, _EPS, 1.0 - _EPS)
        bce = -(onehot * jnp.log(pc) + (1.0 - onehot) * jnp.log(1.0 - pc))
        class_loss = class_loss + has_obj * (
            jnp.sum(wm[:, :, None] * bce) / jnp.maximum(n_obj * _NCLS, 1.0))
    total = (5.0 * coord_loss + obj_loss + 0.5 * noobj_loss + class_loss) / _B
    o_total[...] = jnp.full((1, 1), total, jnp.float32)
    o_coord[...] = jnp.full((1, 1), coord_loss / _B, jnp.float32)
    o_obj[...] = jnp.full((1, 1), obj_loss / _B, jnp.float32)
    o_noobj[...] = jnp.full((1, 1), noobj_loss / _B, jnp.float32)
    o_class[...] = jnp.full((1, 1), class_loss / _B, jnp.float32)


def kernel(pred_s1, pred_s2, pred_s3, boxes, labels):
    p1t, p2t, p3t = (p.transpose(1, 2, 3, 0, 4).reshape(-1, _B, 85)
                     for p in (pred_s1, pred_s2, pred_s3))
    boxes_t = jnp.pad(boxes, ((0, 0), (0, 32 - _NB), (0, 0)),
                      mode="edge").transpose(0, 2, 1)
    rows, meta = _sc_gather(boxes_t, p1t, p2t, p3t)
    parts1, parts2, parts3 = _dense_sum(p1t, p2t, p3t)
    t, c, o, n, cl = _assembly(parts1, parts2, parts3, boxes,
                               labels.astype(jnp.int32), rows, meta)
    return (t.reshape(()), c.reshape(()), o.reshape(()),
            n.reshape(()), cl.reshape(()))
